# baseline (speedup 1.0000x reference)
import functools

import jax
import jax.numpy as jnp
from jax import lax
from jax.experimental import pallas as pl
from jax.experimental.pallas import tpu as pltpu
from jax.experimental.pallas import tpu_sc as plsc

B, T, D = 1024, 50, 64
V_ANS = 1000
V_TAB = V_ANS + 2 * T
NC, NS, L = 2, 16, 16
NW = NC * NS
BPW = B // NW
NPAIR = BPW // 2
ROW_W = T * D


@functools.partial(
    pl.kernel,
    mesh=plsc.VectorSubcoreMesh(core_axis_name="c", subcore_axis_name="s"),
    out_type=jax.ShapeDtypeStruct((B * T * D,), jnp.float32),
    scratch_types=[
        pltpu.VMEM((V_TAB * D,), jnp.float32),
        pltpu.VMEM((BPW * T,), jnp.int32),
        pltpu.VMEM((ROW_W,), jnp.float32),
        pltpu.VMEM((ROW_W,), jnp.float32),
        pltpu.SemaphoreType.DMA,
        pltpu.SemaphoreType.DMA,
        pltpu.SemaphoreType.DMA,
        pltpu.SemaphoreType.DMA,
        pltpu.SemaphoreType.DMA,
        pltpu.SemaphoreType.DMA,
    ],
    compiler_params=pltpu.CompilerParams(
        needs_layout_passes=False, use_tc_tiling_on_sc=False
    ),
)
def _gather_kernel(
    ans_hbm, ocr_hbm, inds_hbm, out_hbm,
    table, idx_all, out0, out1,
    sem_ans, sem_idx, so0, so1, su0, su1,
):
    wid = lax.axis_index("s") * NC + lax.axis_index("c")
    b0 = wid * BPW

    cp_ans = pltpu.async_copy(
        ans_hbm, table.at[pl.ds(0, V_ANS * D)], sem_ans
    )
    cp_idx = pltpu.async_copy(
        inds_hbm.at[pl.ds(b0 * T, BPW * T)], idx_all, sem_idx
    )
    pltpu.async_copy(
        ocr_hbm.at[pl.ds(b0 * ROW_W, ROW_W)],
        table.at[pl.ds(V_ANS * D, ROW_W)], so0,
    )
    pltpu.async_copy(
        ocr_hbm.at[pl.ds((b0 + 1) * ROW_W, ROW_W)],
        table.at[pl.ds((V_ANS + T) * D, ROW_W)], so1,
    )
    cp_idx.wait()
    cp_ans.wait()

    def do_batch(j, i, slot, out_buf, sem_o, sem_u):
        slot_ds = pl.ds((V_ANS + T * slot) * D, ROW_W)
        pltpu.make_async_copy(
            ocr_hbm.at[pl.ds(0, ROW_W)], table.at[slot_ds], sem_o
        ).wait()

        @pl.when(j > 0)
        def _():
            pltpu.make_async_copy(
                out_buf, out_hbm.at[pl.ds(0, ROW_W)], sem_u
            ).wait()

        for r in range(T):
            row = plsc.load_gather(
                idx_all, [jnp.full((L,), i * T + r, jnp.int32)]
            )
            if slot == 1:
                row = jnp.where(row >= V_ANS, row + T, row)
            base = row * D
            for q in range(D // L):
                col = lax.iota(jnp.int32, L) + (L * q)
                out_buf[pl.ds(r * D + L * q, L)] = plsc.load_gather(
                    table, [base + col]
                )

        pltpu.async_copy(out_buf, out_hbm.at[pl.ds((b0 + i) * ROW_W, ROW_W)], sem_u)

        @pl.when(j < NPAIR - 1)
        def _():
            pltpu.async_copy(
                ocr_hbm.at[pl.ds((b0 + i + 2) * ROW_W, ROW_W)],
                table.at[slot_ds], sem_o,
            )

    def pair_step(j, carry):
        do_batch(j, 2 * j, 0, out0, so0, su0)
        do_batch(j, 2 * j + 1, 1, out1, so1, su1)
        return carry

    lax.fori_loop(0, NPAIR, pair_step, 0)
    pltpu.make_async_copy(out0, out_hbm.at[pl.ds(0, ROW_W)], su0).wait()
    pltpu.make_async_copy(out1, out_hbm.at[pl.ds(0, ROW_W)], su1).wait()


def kernel(ans_emb, ocr_emb, prev_inds):
    ans1 = ans_emb.reshape(-1)
    ocr1 = ocr_emb.reshape(-1)
    inds1 = prev_inds.astype(jnp.int32).reshape(-1)
    out1 = _gather_kernel(ans1, ocr1, inds1)
    return out1.reshape(B, T, D)


# speedup vs baseline: 1.0098x; 1.0098x over previous
import functools

import jax
import jax.numpy as jnp
from jax import lax
from jax.experimental import pallas as pl
from jax.experimental.pallas import tpu as pltpu
from jax.experimental.pallas import tpu_sc as plsc

B, T, D = 1024, 50, 64
V_ANS = 1000
V_TAB = V_ANS + 2 * T
NC, NS, L = 2, 16, 16
NW = NC * NS
BPW = B // NW
NPAIR = BPW // 2


@functools.partial(
    pl.kernel,
    mesh=plsc.VectorSubcoreMesh(core_axis_name="c", subcore_axis_name="s"),
    out_type=jax.ShapeDtypeStruct((B, T, D), jnp.float32),
    scratch_types=[
        pltpu.VMEM((V_TAB, D), jnp.float32),
        pltpu.VMEM((BPW, T), jnp.int32),
        pltpu.VMEM((T, D), jnp.float32),
        pltpu.VMEM((T, D), jnp.float32),
        pltpu.SemaphoreType.DMA,
        pltpu.SemaphoreType.DMA,
        pltpu.SemaphoreType.DMA,
        pltpu.SemaphoreType.DMA,
        pltpu.SemaphoreType.DMA,
        pltpu.SemaphoreType.DMA,
    ],
    compiler_params=pltpu.CompilerParams(
        needs_layout_passes=False, use_tc_tiling_on_sc=False
    ),
)
def _gather_kernel(
    ans_hbm, ocr_hbm, inds_hbm, out_hbm,
    table, idx_all, out0, out1,
    sem_ans, sem_idx, so0, so1, su0, su1,
):
    wid = lax.axis_index("s") * NC + lax.axis_index("c")
    b0 = wid * BPW

    cp_ans = pltpu.async_copy(ans_hbm, table.at[pl.ds(0, V_ANS)], sem_ans)
    cp_idx = pltpu.async_copy(inds_hbm.at[pl.ds(b0, BPW)], idx_all, sem_idx)
    pltpu.async_copy(ocr_hbm.at[b0], table.at[pl.ds(V_ANS, T)], so0)
    pltpu.async_copy(ocr_hbm.at[b0 + 1], table.at[pl.ds(V_ANS + T, T)], so1)
    cp_idx.wait()
    cp_ans.wait()

    def do_batch(j, i, slot, out_buf, sem_o, sem_u):
        slot_ds = pl.ds(V_ANS + T * slot, T)
        pltpu.make_async_copy(ocr_hbm.at[b0], table.at[slot_ds], sem_o).wait()

        @pl.when(j > 0)
        def _():
            pltpu.make_async_copy(out_buf, out_hbm.at[b0], sem_u).wait()

        for r in range(T):
            row = plsc.load_gather(
                idx_all,
                [jnp.full((L,), i, jnp.int32), jnp.full((L,), r, jnp.int32)],
            )
            if slot == 1:
                row = jnp.where(row >= V_ANS, row + T, row)
            for q in range(D // L):
                col = lax.iota(jnp.int32, L) + (L * q)
                out_buf[r, pl.ds(L * q, L)] = plsc.load_gather(
                    table, [row, col]
                )

        pltpu.async_copy(out_buf, out_hbm.at[b0 + i], sem_u)

        @pl.when(j < NPAIR - 1)
        def _():
            pltpu.async_copy(ocr_hbm.at[b0 + i + 2], table.at[slot_ds], sem_o)

    def pair_step(j, carry):
        do_batch(j, 2 * j, 0, out0, so0, su0)
        do_batch(j, 2 * j + 1, 1, out1, so1, su1)
        return carry

    lax.fori_loop(0, NPAIR, pair_step, 0)
    pltpu.make_async_copy(out0, out_hbm.at[b0], su0).wait()
    pltpu.make_async_copy(out1, out_hbm.at[b0], su1).wait()


def kernel(ans_emb, ocr_emb, prev_inds):
    return _gather_kernel(ans_emb, ocr_emb, prev_inds.astype(jnp.int32))
